# baseline (speedup 1.0000x reference)
import functools

import jax
import jax.numpy as jnp
from jax import lax
from jax.experimental import pallas as pl
from jax.experimental.pallas import tpu as pltpu
from jax.experimental.pallas import tpu_sc as plsc

N = 10000
D = 128
E = 320000

NSUB = 16
CH = 128
K = 160
E_PAD = NSUB * K * CH
R = E_PAD // CH
NPADROWS = 112
NACC = N + NPADROWS
CNTW = 8
BLK = 8
NB = K // BLK

_f32 = jnp.float32


def _sc_pass_body(with_counts, *refs):
    if with_counts:
        (table, s2d, d2d, zfeat, zcnt, ones_h,
         out_ref, cnt_ref,
         sidx, didx, rows, ones_v, acc, cacc,
         g0, g1, s0, s1, i0, i1, csem) = refs
    else:
        (table, s2d, d2d, zfeat,
         out_ref,
         sidx, didx, rows, acc, g0, g1, s0, s1, i0, i1) = refs
        zcnt = ones_h = cnt_ref = ones_v = cacc = csem = None
    gsems = (g0, g1)
    ssems = (s0, s1)
    isems = (i0, i1)

    s = lax.axis_index("s")
    rpt = NACC // NSUB

    def idx_pair(g, b):
        return (
            pltpu.make_async_copy(s2d.at[pl.ds(s * K + g * BLK, BLK)],
                                  sidx.at[b], isems[b]),
            pltpu.make_async_copy(d2d.at[pl.ds(s * K + g * BLK, BLK)],
                                  didx.at[b], isems[b]),
        )

    def idx_dma(g, b):
        for cp in idx_pair(g, b):
            cp.start()

    def wait_idx(g, b):
        for cp in idx_pair(g, b):
            cp.wait()

    def scat_wait(b):
        pltpu.make_async_copy(rows.at[b], acc.at[didx.at[0, 0]],
                              ssems[b]).wait()

    idx_dma(0, 0)
    if with_counts:
        pltpu.sync_copy(ones_h, ones_v)
    pltpu.sync_copy(zfeat.at[pl.ds(s * rpt, rpt)], acc.at[pl.ds(s * rpt, rpt)])
    if with_counts:
        pltpu.sync_copy(zcnt.at[pl.ds(s * rpt, rpt)],
                        cacc.at[pl.ds(s * rpt, rpt)])
    plsc.subcore_barrier()

    def block(g, bi):
        wait_idx(g, bi)

        @pl.when(g + 1 < NB)
        def _():
            idx_dma(g + 1, 1 - bi)

        def gather(t, bt):
            pltpu.async_copy(table.at[sidx.at[bi, t]], rows.at[bt],
                             gsems[bt])

        @pl.when(g > 0)
        def _():
            scat_wait(0)
        gather(0, 0)

        for t in range(BLK):
            bt = t % 2
            pltpu.make_async_copy(table.at[sidx.at[bi, t]], rows.at[bt],
                                  gsems[bt]).wait()
            pltpu.async_copy(rows.at[bt], acc.at[didx.at[bi, t]], ssems[bt],
                             add=True)
            if with_counts:
                jglob = g * BLK + t

                @pl.when(jglob > 0)
                def _():
                    pltpu.make_async_copy(ones_v, cacc.at[didx.at[bi, t]],
                                          csem).wait()
                pltpu.async_copy(ones_v, cacc.at[didx.at[bi, t]], csem,
                                 add=True)
            if t < BLK - 1:
                nbt = (t + 1) % 2
                if t == 0:
                    @pl.when(g > 0)
                    def _():
                        scat_wait(nbt)
                else:
                    scat_wait(nbt)
                gather(t + 1, nbt)

    def step(g2, _):
        block(2 * g2, 0)
        block(2 * g2 + 1, 1)
        return 0

    lax.fori_loop(0, NB // 2, step, 0)

    scat_wait(0)
    scat_wait(1)
    if with_counts:
        pltpu.make_async_copy(ones_v, cacc.at[didx.at[0, 0]], csem).wait()

    plsc.subcore_barrier()
    pltpu.sync_copy(acc.at[pl.ds(s * rpt, rpt)],
                    out_ref.at[pl.ds(s * rpt, rpt)])
    if with_counts:
        pltpu.sync_copy(cacc.at[pl.ds(s * rpt, rpt)],
                        cnt_ref.at[pl.ds(s * rpt, rpt)])


def _make_sc_pass(with_counts):
    out_type = [jax.ShapeDtypeStruct((NACC, D), _f32)]
    scratch = [
        pltpu.VMEM((2, BLK, CH), jnp.int32),
        pltpu.VMEM((2, BLK, CH), jnp.int32),
        pltpu.VMEM((2, CH, D), _f32),
    ]
    if with_counts:
        out_type.append(jax.ShapeDtypeStruct((NACC, CNTW), _f32))
        scratch.append(pltpu.VMEM((CH, CNTW), _f32))
    scratch.append(pltpu.VMEM_SHARED((NACC, D), _f32))
    if with_counts:
        scratch.append(pltpu.VMEM_SHARED((NACC, CNTW), _f32))
    scratch += [pltpu.SemaphoreType.DMA] * 6
    if with_counts:
        scratch.append(pltpu.SemaphoreType.DMA)

    return pl.kernel(
        functools.partial(_sc_pass_body, with_counts),
        out_type=out_type,
        mesh=plsc.VectorSubcoreMesh(core_axis_name="c", subcore_axis_name="s",
                                    num_cores=1),
        scratch_types=scratch,
        compiler_params=pltpu.CompilerParams(use_tc_tiling_on_sc=False),
        name="sage_segsum_cnt" if with_counts else "sage_segsum",
    )


def _dotT(x, w):
    return lax.dot_general(x, w, (((1,), (1,)), ((), ())),
                           precision=lax.Precision.HIGHEST,
                           preferred_element_type=_f32)


_TCB = 1000


def _row_spec():
    return pl.BlockSpec((_TCB, D), lambda i: (i, 0))


def _cnt_spec():
    return pl.BlockSpec((_TCB, CNTW), lambda i: (i, 0))


def _full_spec():
    return pl.BlockSpec((D, D), lambda i: (0, 0))


def _bias_spec():
    return pl.BlockSpec((1, D), lambda i: (0, 0))


def _proj_body(xu, xi, pu, pi, hu, hi):
    hu[...] = _dotT(xu[...], pu[...])
    hi[...] = _dotT(xi[...], pi[...])


def _tc_proj(x_u, x_i, P_u, P_i):
    return pl.pallas_call(
        _proj_body,
        grid=(N // _TCB,),
        in_specs=[_row_spec(), _row_spec(), _full_spec(), _full_spec()],
        out_specs=[_row_spec(), _row_spec()],
        out_shape=[jax.ShapeDtypeStruct((N, D), _f32)] * 2,
    )(x_u, x_i, P_u, P_i)


def _update(aggr, cnt, x_dst, wl, bl, wr):
    inv = 1.0 / jnp.clip(cnt[:, 0:1], 1.0)
    return _dotT(aggr, wl) * inv + bl + _dotT(x_dst, wr)


def _layer1_body(ai, ci, hi, wli, bli, wri, au, cu, hu, wlu, blu, wru,
                 oi, ou):
    pre_i = _update(ai[...], ci[...], hi[...], wli[...], bli[...], wri[...])
    pre_u = _update(au[...], cu[...], hu[...], wlu[...], blu[...], wru[...])
    oi[...] = jnp.maximum(pre_i, 0.0) + hi[...]
    ou[...] = jnp.maximum(pre_u, 0.0) + hu[...]


def _side_specs():
    return [_row_spec(), _cnt_spec(), _row_spec(),
            _full_spec(), _bias_spec(), _full_spec()]


def _tc_layer1(*args):
    return pl.pallas_call(
        _layer1_body,
        grid=(N // _TCB,),
        in_specs=_side_specs() + _side_specs(),
        out_specs=[_row_spec(), _row_spec()],
        out_shape=[jax.ShapeDtypeStruct((N, D), _f32)] * 2,
    )(*args)


def _layer2_body(au, cu, ou, wlu, blu, wru, ai, ci, oi, wli, bli, wri,
                 out_u, out_i):
    p_u = _update(au[...], cu[...], ou[...], wlu[...], blu[...], wru[...])
    p_i = _update(ai[...], ci[...], oi[...], wli[...], bli[...], wri[...])
    n_u = jnp.sqrt(jnp.sum(p_u * p_u, axis=1, keepdims=True))
    n_i = jnp.sqrt(jnp.sum(p_i * p_i, axis=1, keepdims=True))
    out_u[...] = p_u / jnp.clip(n_u, 1e-12)
    out_i[...] = p_i / jnp.clip(n_i, 1e-12)


def _tc_layer2(*args):
    return pl.pallas_call(
        _layer2_body,
        grid=(N // _TCB,),
        in_specs=_side_specs() + _side_specs(),
        out_specs=[_row_spec(), _row_spec()],
        out_shape=[jax.ShapeDtypeStruct((N, D), _f32)] * 2,
    )(*args)


def _prep_edges(ei):
    pad = E_PAD - E
    src = jnp.concatenate([ei[0], jnp.zeros((pad,), jnp.int32)])
    pad_dst = N + (jnp.arange(pad, dtype=jnp.int32) % NPADROWS)
    dst = jnp.concatenate([ei[1], pad_dst])
    return src.reshape(R, CH), dst.reshape(R, CH)


def kernel(x_user, x_item, edge_index_ui, edge_index_iu, P_user, P_item,
           c1_ui_Wl, c1_ui_bl, c1_ui_Wr, c1_iu_Wl, c1_iu_bl, c1_iu_Wr,
           c2_ui_Wl, c2_ui_bl, c2_ui_Wr, c2_iu_Wl, c2_iu_bl, c2_iu_Wr):
    s_ui, d_ui = _prep_edges(edge_index_ui)
    s_iu, d_iu = _prep_edges(edge_index_iu)
    zfeat = jnp.zeros((NACC, D), _f32)
    zcnt = jnp.zeros((NACC, CNTW), _f32)
    ones_h = jnp.ones((CH, CNTW), _f32)
    b1_ui = c1_ui_bl.reshape(1, D)
    b1_iu = c1_iu_bl.reshape(1, D)
    b2_ui = c2_ui_bl.reshape(1, D)
    b2_iu = c2_iu_bl.reshape(1, D)

    h_u, h_i = _tc_proj(x_user, x_item, P_user, P_item)

    sc_cnt = _make_sc_pass(True)
    sc_plain = _make_sc_pass(False)

    a1_i, cnt_i = sc_cnt(h_u, s_ui, d_ui, zfeat, zcnt, ones_h)
    a1_u, cnt_u = sc_cnt(h_i, s_iu, d_iu, zfeat, zcnt, ones_h)

    o_i, o_u = _tc_layer1(a1_i, cnt_i, h_i, c1_ui_Wl, b1_ui, c1_ui_Wr,
                          a1_u, cnt_u, h_u, c1_iu_Wl, b1_iu, c1_iu_Wr)

    a2_i = sc_plain(o_u, s_ui, d_ui, zfeat)[0]
    a2_u = sc_plain(o_i, s_iu, d_iu, zfeat)[0]

    out_u, out_i = _tc_layer2(a2_u, cnt_u, o_u, c2_iu_Wl, b2_iu, c2_iu_Wr,
                              a2_i, cnt_i, o_i, c2_ui_Wl, b2_ui, c2_ui_Wr)
    return (out_u, out_i)


# speedup vs baseline: 1.9126x; 1.9126x over previous
import functools

import jax
import jax.numpy as jnp
from jax import lax
from jax.experimental import pallas as pl
from jax.experimental.pallas import tpu as pltpu
from jax.experimental.pallas import tpu_sc as plsc

N = 10000
D = 128
HD = D // 2
E = 320000

NSUB = 16
CH = 128
K = 160
E_PAD = NSUB * K * CH
R = E_PAD // CH
NPADROWS = 112
NACC = N + NPADROWS
CNTW = 8
NG = 4
NS = 2

_f32 = jnp.float32
_bf16 = jnp.bfloat16

_PERM64 = ([*range(0, 32, 2)] + [*range(1, 32, 2)]
           + [*range(32, 64, 2)] + [*range(33, 64, 2)])
_PERM = _PERM64 + [64 + k for k in _PERM64]


def _sc_pass_body(with_counts, *refs):
    if with_counts:
        (ta0, ta1, tb0, tb1, s_a, d_a, s_b, d_b, zfeat, zcnt, ones_h,
         oa0, oa1, ob0, ob1, cnt_a, cnt_b,
         sidx, didx, rbf, ones_v, fbuf, acc, cacc, *sems) = refs
        csem = sems[NG + NS]
    else:
        (ta0, ta1, tb0, tb1, s_a, d_a, s_b, d_b, zfeat,
         oa0, oa1, ob0, ob1,
         sidx, didx, rbf, fbuf, acc, *sems) = refs
        zcnt = ones_h = cnt_a = cnt_b = ones_v = cacc = csem = None
    gsems = tuple(sems[:NG])
    ssems = tuple(sems[NG:NG + NS])

    c = lax.axis_index("c")
    s = lax.axis_index("s")
    rpt = NACC // NSUB

    def run(tabs, s2d, d2d, outs, cnt_ref):
        pltpu.sync_copy(s2d.at[pl.ds(s * K, K)], sidx)
        pltpu.sync_copy(d2d.at[pl.ds(s * K, K)], didx)
        if with_counts:
            pltpu.sync_copy(ones_h, ones_v)

        for p in range(2):
            table = tabs[p]
            do_cnt = with_counts and p == 0
            pltpu.sync_copy(zfeat.at[pl.ds(s * rpt, rpt)],
                            acc.at[pl.ds(s * rpt, rpt)])
            if do_cnt:
                pltpu.sync_copy(zcnt.at[pl.ds(s * rpt, rpt)],
                                cacc.at[pl.ds(s * rpt, rpt)])
            plsc.subcore_barrier()

            def gather(j, b):
                pltpu.async_copy(table.at[sidx.at[j]], rbf.at[b], gsems[b])

            gather(0, 0)
            gather(1, 1)

            def step(g, _):
                for u in range(4):
                    j = 4 * g + u
                    bg = u % NG
                    bs = u % NS
                    pltpu.make_async_copy(table.at[sidx.at[j]], rbf.at[bg],
                                          gsems[bg]).wait()

                    @pl.when(j + 2 < K)
                    def _():
                        gather(j + 2, (u + 2) % NG)

                    @pl.when(j >= NS)
                    def _():
                        pltpu.make_async_copy(fbuf.at[bs],
                                              acc.at[didx.at[j]],
                                              ssems[bs]).wait()

                    def conv(r, _):
                        x0 = rbf[bg, r, pl.ds(0, 32)]
                        e0, o0 = plsc.unpack(
                            x0, format=plsc.PackFormat.INTERLEAVED)
                        fbuf[bs, r, pl.ds(0, 16)] = e0
                        fbuf[bs, r, pl.ds(16, 16)] = o0
                        x1 = rbf[bg, r, pl.ds(32, 32)]
                        e1, o1 = plsc.unpack(
                            x1, format=plsc.PackFormat.INTERLEAVED)
                        fbuf[bs, r, pl.ds(32, 16)] = e1
                        fbuf[bs, r, pl.ds(48, 16)] = o1
                        return 0

                    lax.fori_loop(0, CH, conv, 0)

                    pltpu.async_copy(fbuf.at[bs], acc.at[didx.at[j]],
                                     ssems[bs], add=True)
                    if do_cnt:
                        @pl.when(j > 0)
                        def _():
                            pltpu.make_async_copy(
                                ones_v, cacc.at[didx.at[j]], csem).wait()
                        pltpu.async_copy(ones_v, cacc.at[didx.at[j]], csem,
                                         add=True)
                return 0

            lax.fori_loop(0, K // 4, step, 0)

            for bs in range(NS):
                pltpu.make_async_copy(fbuf.at[bs], acc.at[didx.at[0]],
                                      ssems[bs]).wait()
            if do_cnt:
                pltpu.make_async_copy(ones_v, cacc.at[didx.at[0]],
                                      csem).wait()

            plsc.subcore_barrier()
            pltpu.sync_copy(acc.at[pl.ds(s * rpt, rpt)],
                            outs[p].at[pl.ds(s * rpt, rpt)])
            if do_cnt:
                pltpu.sync_copy(cacc.at[pl.ds(s * rpt, rpt)],
                                cnt_ref.at[pl.ds(s * rpt, rpt)])
            plsc.subcore_barrier()

    @pl.when(c == 0)
    def _():
        run((ta0, ta1), s_a, d_a, (oa0, oa1), cnt_a)

    @pl.when(c == 1)
    def _():
        run((tb0, tb1), s_b, d_b, (ob0, ob1), cnt_b)


def _make_sc_pass(with_counts):
    out_type = [jax.ShapeDtypeStruct((NACC, HD), _f32)] * 4
    scratch = [
        pltpu.VMEM((K, CH), jnp.int32),
        pltpu.VMEM((K, CH), jnp.int32),
        pltpu.VMEM((NG, CH, HD), _bf16),
    ]
    if with_counts:
        out_type += [jax.ShapeDtypeStruct((NACC, CNTW), _f32)] * 2
        scratch.append(pltpu.VMEM((CH, CNTW), _f32))
    scratch.append(pltpu.VMEM((NS, CH, HD), _f32))
    scratch.append(pltpu.VMEM_SHARED((NACC, HD), _f32))
    if with_counts:
        scratch.append(pltpu.VMEM_SHARED((NACC, CNTW), _f32))
    scratch += [pltpu.SemaphoreType.DMA] * (NG + NS)
    if with_counts:
        scratch.append(pltpu.SemaphoreType.DMA)

    return pl.kernel(
        functools.partial(_sc_pass_body, with_counts),
        out_type=out_type,
        mesh=plsc.VectorSubcoreMesh(core_axis_name="c", subcore_axis_name="s"),
        scratch_types=scratch,
        compiler_params=pltpu.CompilerParams(use_tc_tiling_on_sc=False,
                                             needs_layout_passes=False),
        name="sage_segsum_cnt" if with_counts else "sage_segsum",
    )


def _dotT(x, w):
    return lax.dot_general(x, w, (((1,), (1,)), ((), ())),
                           precision=lax.Precision.HIGHEST,
                           preferred_element_type=_f32)


_TCB = 1000


def _row_spec(w):
    return pl.BlockSpec((_TCB, w), lambda i: (i, 0))


def _full_spec():
    return pl.BlockSpec((D, D), lambda i: (0, 0))


def _bias_spec():
    return pl.BlockSpec((1, D), lambda i: (0, 0))


def _proj_body(xu, xi, pu, pi, hu0, hu1, hi0, hi1, bu0, bu1, bi0, bi1):
    hu = _dotT(xu[...], pu[...])
    hi = _dotT(xi[...], pi[...])
    hu0[...] = hu[:, :HD]
    hu1[...] = hu[:, HD:]
    hi0[...] = hi[:, :HD]
    hi1[...] = hi[:, HD:]
    bu0[...] = hu[:, :HD].astype(_bf16)
    bu1[...] = hu[:, HD:].astype(_bf16)
    bi0[...] = hi[:, :HD].astype(_bf16)
    bi1[...] = hi[:, HD:].astype(_bf16)


def _tc_proj(x_u, x_i, P_u, P_i):
    return pl.pallas_call(
        _proj_body,
        grid=(N // _TCB,),
        in_specs=[_row_spec(D), _row_spec(D), _full_spec(), _full_spec()],
        out_specs=[_row_spec(HD)] * 8,
        out_shape=[jax.ShapeDtypeStruct((N, HD), _f32)] * 4
        + [jax.ShapeDtypeStruct((N, HD), _bf16)] * 4,
    )(x_u, x_i, P_u, P_i)


def _halved_update(a0, a1, cnt, x0, x1, wl, bl, wr):
    inv = 1.0 / jnp.clip(cnt[:, 0:1], 1.0)
    aw = _dotT(a0, wl[:, :HD]) + _dotT(a1, wl[:, HD:])
    xw = _dotT(x0, wr[:, :HD]) + _dotT(x1, wr[:, HD:])
    return aw * inv + bl + xw


def _layer1_body(ai0, ai1, ci, hi0, hi1, wli, bli, wri,
                 au0, au1, cu, hu0, hu1, wlu, blu, wru,
                 oi0, oi1, ou0, ou1, qi0, qi1, qu0, qu1):
    pre_i = _halved_update(ai0[...], ai1[...], ci[...], hi0[...], hi1[...],
                           wli[...], bli[...], wri[...])
    pre_u = _halved_update(au0[...], au1[...], cu[...], hu0[...], hu1[...],
                           wlu[...], blu[...], wru[...])
    oi0[...] = jnp.maximum(pre_i[:, :HD], 0.0) + hi0[...]
    oi1[...] = jnp.maximum(pre_i[:, HD:], 0.0) + hi1[...]
    ou0[...] = jnp.maximum(pre_u[:, :HD], 0.0) + hu0[...]
    ou1[...] = jnp.maximum(pre_u[:, HD:], 0.0) + hu1[...]
    qi0[...] = oi0[...].astype(_bf16)
    qi1[...] = oi1[...].astype(_bf16)
    qu0[...] = ou0[...].astype(_bf16)
    qu1[...] = ou1[...].astype(_bf16)


def _side_specs():
    return [_row_spec(HD), _row_spec(HD), pl.BlockSpec((_TCB, CNTW), lambda i: (i, 0)),
            _row_spec(HD), _row_spec(HD), _full_spec(), _bias_spec(), _full_spec()]


def _tc_layer1(*args):
    return pl.pallas_call(
        _layer1_body,
        grid=(N // _TCB,),
        in_specs=_side_specs() + _side_specs(),
        out_specs=[_row_spec(HD)] * 8,
        out_shape=[jax.ShapeDtypeStruct((N, HD), _f32)] * 4
        + [jax.ShapeDtypeStruct((N, HD), _bf16)] * 4,
    )(*args)


def _layer2_body(au0, au1, cu, ou0, ou1, wlu, blu, wru,
                 ai0, ai1, ci, oi0, oi1, wli, bli, wri,
                 out_u, out_i):
    p_u = _halved_update(au0[...], au1[...], cu[...], ou0[...], ou1[...],
                         wlu[...], blu[...], wru[...])
    p_i = _halved_update(ai0[...], ai1[...], ci[...], oi0[...], oi1[...],
                         wli[...], bli[...], wri[...])
    n_u = jnp.sqrt(jnp.sum(p_u * p_u, axis=1, keepdims=True))
    n_i = jnp.sqrt(jnp.sum(p_i * p_i, axis=1, keepdims=True))
    out_u[...] = p_u / jnp.clip(n_u, 1e-12)
    out_i[...] = p_i / jnp.clip(n_i, 1e-12)


def _tc_layer2(*args):
    return pl.pallas_call(
        _layer2_body,
        grid=(N // _TCB,),
        in_specs=_side_specs() + _side_specs(),
        out_specs=[_row_spec(D)] * 2,
        out_shape=[jax.ShapeDtypeStruct((N, D), _f32)] * 2,
    )(*args)


def _prep_edges(ei):
    pad = E_PAD - E
    src = jnp.concatenate([ei[0], jnp.zeros((pad,), jnp.int32)])
    pad_dst = N + (jnp.arange(pad, dtype=jnp.int32) % NPADROWS)
    dst = jnp.concatenate([ei[1], pad_dst])
    return src.reshape(R, CH), dst.reshape(R, CH)


def kernel(x_user, x_item, edge_index_ui, edge_index_iu, P_user, P_item,
           c1_ui_Wl, c1_ui_bl, c1_ui_Wr, c1_iu_Wl, c1_iu_bl, c1_iu_Wr,
           c2_ui_Wl, c2_ui_bl, c2_ui_Wr, c2_iu_Wl, c2_iu_bl, c2_iu_Wr):
    s_ui, d_ui = _prep_edges(edge_index_ui)
    s_iu, d_iu = _prep_edges(edge_index_iu)
    zfeat = jnp.zeros((NACC, HD), _f32)
    zcnt = jnp.zeros((NACC, CNTW), _f32)
    ones_h = jnp.ones((CH, CNTW), _f32)
    b1_ui = c1_ui_bl.reshape(1, D)
    b1_iu = c1_iu_bl.reshape(1, D)
    b2_ui = c2_ui_bl.reshape(1, D)
    b2_iu = c2_iu_bl.reshape(1, D)

    perm = jnp.array(_PERM, jnp.int32)
    wl1_ui = c1_ui_Wl[:, perm]
    wl1_iu = c1_iu_Wl[:, perm]
    wl2_ui = c2_ui_Wl[:, perm]
    wl2_iu = c2_iu_Wl[:, perm]

    hu0, hu1, hi0, hi1, bu0, bu1, bi0, bi1 = _tc_proj(
        x_user, x_item, P_user, P_item)

    sc_cnt = _make_sc_pass(True)
    sc_plain = _make_sc_pass(False)

    a1i0, a1i1, a1u0, a1u1, cnt_i, cnt_u = sc_cnt(
        bu0, bu1, bi0, bi1, s_ui, d_ui, s_iu, d_iu, zfeat, zcnt, ones_h)

    oi0, oi1, ou0, ou1, qi0, qi1, qu0, qu1 = _tc_layer1(
        a1i0, a1i1, cnt_i, hi0, hi1, wl1_ui, b1_ui, c1_ui_Wr,
        a1u0, a1u1, cnt_u, hu0, hu1, wl1_iu, b1_iu, c1_iu_Wr)

    a2i0, a2i1, a2u0, a2u1 = sc_plain(
        qu0, qu1, qi0, qi1, s_ui, d_ui, s_iu, d_iu, zfeat)

    out_u, out_i = _tc_layer2(
        a2u0, a2u1, cnt_u, ou0, ou1, wl2_iu, b2_iu, c2_iu_Wr,
        a2i0, a2i1, cnt_i, oi0, oi1, wl2_ui, b2_ui, c2_ui_Wr)
    return (out_u, out_i)
